# baseline (speedup 1.0000x reference)
import functools

import jax
import jax.numpy as jnp
import numpy as np
from jax import lax
from jax.experimental import pallas as pl
from jax.experimental.pallas import tpu as pltpu
from jax.experimental.pallas import tpu_sc as plsc

N = 10000
E = 320000
D = 128

NC = 2
NS = 16
NW = NC * NS
CHUNK = 64
G = 4
EPW = 10240
EPAD = NW * EPW
NCHUNK = EPW // CHUNK
NSUP = NCHUNK // G
ROWS_BASE = 624
ROWS_LAST = N - 15 * ROWS_BASE
NZFULL = N // CHUNK
NZTAIL = N - NZFULL * CHUNK

_PERM = np.empty(D, np.int32)
for _g in range(D // 32):
    for _i in range(16):
        _PERM[32 * _g + 2 * _i] = 32 * _g + _i
        _PERM[32 * _g + 2 * _i + 1] = 32 * _g + 16 + _i


def _sc_body(combo_hbm, ev_hbm, featsb_hbm, out_hbm,
             ibuf, ebuf, gbufb, sbuf, acc,
             isem0, isem1, esem0, esem1, gsem0, gsem1, ssem0, ssem1):
    c = lax.axis_index("c")
    s = lax.axis_index("s")
    gw = c * NS + s
    isem = (isem0, isem1)
    esem = (esem0, esem1)
    gsem = (gsem0, gsem1)
    ssem = (ssem0, ssem1)

    ld_i = pltpu.async_copy(combo_hbm.at[gw, 0], ibuf.at[0], isem0)
    pltpu.async_copy(combo_hbm.at[gw, 1], ibuf.at[1], isem1)
    ld_e = pltpu.async_copy(ev_hbm.at[gw, 0], ebuf.at[0], esem0)
    pltpu.async_copy(ev_hbm.at[gw, 1], ebuf.at[1], esem1)

    def zrow(i, _):
        for j in range(D // 16):
            sbuf[0, i, pl.ds(j * 16, 16)] = jnp.zeros((16,), jnp.float32)
        return 0
    lax.fori_loop(0, CHUNK, zrow, 0)
    for t in range(10):
        zk = s * 10 + t

        @pl.when(zk < NZFULL)
        def _():
            off = pl.multiple_of(zk * CHUNK, 8)
            pltpu.sync_copy(sbuf.at[0], acc.at[pl.ds(off, CHUNK)])

    @pl.when(s == 15)
    def _():
        pltpu.sync_copy(sbuf.at[0, pl.ds(0, NZTAIL)],
                        acc.at[pl.ds(NZFULL * CHUNK, NZTAIL)])

    ld_i.wait()
    ld_e.wait()
    pltpu.async_copy(featsb_hbm.at[ibuf.at[0, 0, 0]], gbufb.at[0], gsem0)
    pltpu.async_copy(featsb_hbm.at[ibuf.at[0, 0, 1]], gbufb.at[1], gsem1)
    plsc.subcore_barrier()

    def step(sc, q, t):
        k = G * sc + t
        p = t % 2

        pltpu.make_async_copy(featsb_hbm.at[ibuf.at[q, 0, t]], gbufb.at[p],
                              gsem[p]).wait()

        if t >= 2:
            d_q, d_t = q, t - 2
        else:
            d_q, d_t = 1 - q, t + 2

        @pl.when(k >= 2)
        def _():
            pltpu.make_async_copy(sbuf.at[p], acc.at[ibuf.at[d_q, 1, d_t]],
                                  ssem[p]).wait()

        if t == 1:
            @pl.when(jnp.logical_and(sc >= 1, sc + 1 < NSUP))
            def _():
                pltpu.async_copy(combo_hbm.at[gw, sc + 1], ibuf.at[1 - q],
                                 isem[1 - q])
                pltpu.async_copy(ev_hbm.at[gw, sc + 1], ebuf.at[1 - q],
                                 esem[1 - q])
        if t == 2:
            @pl.when(sc + 1 < NSUP)
            def _():
                pltpu.make_async_copy(combo_hbm.at[gw, sc + 1],
                                      ibuf.at[1 - q], isem[1 - q]).wait()
                pltpu.make_async_copy(ev_hbm.at[gw, sc + 1],
                                      ebuf.at[1 - q], esem[1 - q]).wait()

        def scale(g, _):
            ev16 = ebuf[q, t, pl.ds(pl.multiple_of(g * 16, 8), 16)]
            for e in range(16):
                evb = jnp.full((16,), ev16[e], jnp.float32)
                r = g * 16 + e
                for h in range(D // 32):
                    xi = gbufb[p, r, pl.ds(h * 16, 16)]
                    x32 = plsc.bitcast(xi, jnp.bfloat16)
                    a, b = plsc.unpack(x32, format=plsc.PackFormat.INTERLEAVED,
                                       preferred_element_type=jnp.float32)
                    sbuf[p, r, pl.ds(h * 32, 16)] = a * evb
                    sbuf[p, r, pl.ds(h * 32 + 16, 16)] = b * evb
            return 0
        lax.fori_loop(0, CHUNK // 16, scale, 0)

        if t < 2:
            g_q, g_t = q, t + 2
        else:
            g_q, g_t = 1 - q, t - 2

        @pl.when(k + 2 < NCHUNK)
        def _():
            pltpu.async_copy(featsb_hbm.at[ibuf.at[g_q, 0, g_t]],
                             gbufb.at[p], gsem[p])

        pltpu.async_copy(sbuf.at[p], acc.at[ibuf.at[q, 1, t]], ssem[p],
                         add=True)

    def super_pair(scp, _):
        for t in range(G):
            step(2 * scp, 0, t)
        for t in range(G):
            step(2 * scp + 1, 1, t)
        return 0
    lax.fori_loop(0, NSUP // 2, super_pair, 0)

    pltpu.make_async_copy(sbuf.at[0], acc.at[ibuf.at[1, 1, 2]],
                          ssem0).wait()
    pltpu.make_async_copy(sbuf.at[1], acc.at[ibuf.at[1, 1, 3]],
                          ssem1).wait()
    plsc.subcore_barrier()

    @pl.when(s < 15)
    def _():
        off = pl.multiple_of(s * ROWS_BASE, 8)
        pltpu.sync_copy(acc.at[pl.ds(off, ROWS_BASE)],
                        out_hbm.at[c, pl.ds(off, ROWS_BASE)])

    @pl.when(s == 15)
    def _():
        off = 15 * ROWS_BASE
        pltpu.sync_copy(acc.at[pl.ds(off, ROWS_LAST)],
                        out_hbm.at[c, pl.ds(off, ROWS_LAST)])


_sc_segment = functools.partial(
    pl.kernel,
    out_type=jax.ShapeDtypeStruct((NC, N, D), jnp.float32),
    mesh=plsc.VectorSubcoreMesh(core_axis_name="c", subcore_axis_name="s"),
    compiler_params=pltpu.CompilerParams(needs_layout_passes=False, use_tc_tiling_on_sc=False),
    scratch_types=[
        pltpu.VMEM((2, 2, G, CHUNK), jnp.int32),
        pltpu.VMEM((2, G, CHUNK), jnp.float32),
        pltpu.VMEM((2, CHUNK, D // 2), jnp.int32),
        pltpu.VMEM((2, CHUNK, D), jnp.float32),
        pltpu.VMEM_SHARED((N, D), jnp.float32),
        pltpu.SemaphoreType.DMA,
        pltpu.SemaphoreType.DMA,
        pltpu.SemaphoreType.DMA,
        pltpu.SemaphoreType.DMA,
        pltpu.SemaphoreType.DMA,
        pltpu.SemaphoreType.DMA,
        pltpu.SemaphoreType.DMA,
        pltpu.SemaphoreType.DMA,
    ],
)(_sc_body)


def _tc_body(lep_ref, f_ref, w1_ref, w2_ref, b1_ref, b2_ref, o_ref):
    le = lep_ref[0] + lep_ref[1]
    f = f_ref[...]
    sf = le + f
    em = le * f
    acc = lax.dot_general(sf, w1_ref[...], (((1,), (1,)), ((), ())),
                          preferred_element_type=jnp.float32)
    acc = acc + lax.dot_general(em, w2_ref[...], (((1,), (1,)), ((), ())),
                                preferred_element_type=jnp.float32)
    o_ref[...] = acc + b1_ref[...] + b2_ref[...]


_BN = 1000


def _tc_dense(lep, feats, W1_w, W1_b, W2_w, W2_b):
    return pl.pallas_call(
        _tc_body,
        grid=(N // _BN,),
        in_specs=[
            pl.BlockSpec((NC, _BN, D), lambda i: (0, i, 0)),
            pl.BlockSpec((_BN, D), lambda i: (i, 0)),
            pl.BlockSpec((D, D), lambda i: (0, 0)),
            pl.BlockSpec((D, D), lambda i: (0, 0)),
            pl.BlockSpec((1, D), lambda i: (0, 0)),
            pl.BlockSpec((1, D), lambda i: (0, 0)),
        ],
        out_specs=pl.BlockSpec((_BN, D), lambda i: (i, 0)),
        out_shape=jax.ShapeDtypeStruct((N, D), jnp.float32),
    )(lep, feats, W1_w, W2_w, W1_b.reshape(1, D), W2_b.reshape(1, D))


def kernel(edge_index, edge_values, feats, W1_w, W1_b, W2_w, W2_b):
    pad = EPAD - E
    spread = (jnp.arange(pad, dtype=jnp.int32) * 8) % N
    src = jnp.concatenate([edge_index[0], spread])
    dst = jnp.concatenate([edge_index[1], spread])
    ev = jnp.concatenate([edge_values, jnp.zeros((pad,), jnp.float32)])
    shp = (NW, NSUP, 1, G, CHUNK)
    combo = jnp.concatenate([src.reshape(shp), dst.reshape(shp)], axis=2)
    evr = ev.reshape(NW, NSUP, G, CHUNK)
    featsb = lax.bitcast_convert_type(
        feats.astype(jnp.bfloat16)[:, _PERM].reshape(N, D // 2, 2),
        jnp.int32)
    lep = _sc_segment(combo, evr, featsb)
    return _tc_dense(lep, feats, W1_w, W1_b, W2_w, W2_b)


# speedup vs baseline: 2.0143x; 2.0143x over previous
import functools

import jax
import jax.numpy as jnp
from jax import lax
from jax.experimental import pallas as pl
from jax.experimental.pallas import tpu as pltpu
from jax.experimental.pallas import tpu_sc as plsc

N = 10000
E = 320000
D = 128

NC = 2
NS = 16
NW = NC * NS
CHUNK = 64
G = 4
EPW = 10240
EPAD = NW * EPW
NCHUNK = EPW // CHUNK
NSUP = NCHUNK // G
ROWS_BASE = 624
ROWS_LAST = N - 15 * ROWS_BASE
NZFULL = N // CHUNK
NZTAIL = N - NZFULL * CHUNK


def _sc_body(combo_hbm, ev_hbm, feats_hbm, out_hbm,
             ibuf, ebuf, gbuf, acc,
             isem0, isem1, esem0, esem1,
             gsem0, gsem1, gsem2, gsem3, ssem0, ssem1, ssem2, ssem3):
    c = lax.axis_index("c")
    s = lax.axis_index("s")
    gw = c * NS + s
    isem = (isem0, isem1)
    esem = (esem0, esem1)
    gsem = (gsem0, gsem1, gsem2, gsem3)
    ssem = (ssem0, ssem1, ssem2, ssem3)

    ld_i = pltpu.async_copy(combo_hbm.at[gw, 0], ibuf.at[0], isem0)
    pltpu.async_copy(combo_hbm.at[gw, 1], ibuf.at[1], isem1)
    ld_e = pltpu.async_copy(ev_hbm.at[gw, 0], ebuf.at[0], esem0)
    pltpu.async_copy(ev_hbm.at[gw, 1], ebuf.at[1], esem1)

    def zrow(i, _):
        for j in range(D // 16):
            gbuf[0, i, pl.ds(j * 16, 16)] = jnp.zeros((16,), jnp.float32)
        return 0
    lax.fori_loop(0, CHUNK, zrow, 0)
    for t in range(10):
        zk = s * 10 + t

        @pl.when(zk < NZFULL)
        def _():
            off = pl.multiple_of(zk * CHUNK, 8)
            pltpu.sync_copy(gbuf.at[0], acc.at[pl.ds(off, CHUNK)])

    @pl.when(s == 15)
    def _():
        pltpu.sync_copy(gbuf.at[0, pl.ds(0, NZTAIL)],
                        acc.at[pl.ds(NZFULL * CHUNK, NZTAIL)])

    ld_i.wait()
    ld_e.wait()
    pltpu.async_copy(feats_hbm.at[ibuf.at[0, 0, 0]], gbuf.at[0], gsem0)
    pltpu.async_copy(feats_hbm.at[ibuf.at[0, 0, 1]], gbuf.at[1], gsem1)
    pltpu.async_copy(feats_hbm.at[ibuf.at[0, 0, 2]], gbuf.at[2], gsem2)
    plsc.subcore_barrier()

    def step(sc, q, t):
        k = G * sc + t

        pltpu.make_async_copy(feats_hbm.at[ibuf.at[q, 0, t]], gbuf.at[t],
                              gsem[t]).wait()

        if t >= 1:
            d_q, d_t = q, t - 1
        else:
            d_q, d_t = 1 - q, 3

        @pl.when(k >= 1)
        def _():
            pltpu.make_async_copy(gbuf.at[(t + 3) % 4],
                                  acc.at[ibuf.at[d_q, 1, d_t]],
                                  ssem[(t + 3) % 4]).wait()

        if t == 0:
            @pl.when(jnp.logical_and(sc >= 1, sc + 1 < NSUP))
            def _():
                pltpu.async_copy(combo_hbm.at[gw, sc + 1], ibuf.at[1 - q],
                                 isem[1 - q])
                pltpu.async_copy(ev_hbm.at[gw, sc + 1], ebuf.at[1 - q],
                                 esem[1 - q])
        if t == 1:
            @pl.when(sc + 1 < NSUP)
            def _():
                pltpu.make_async_copy(combo_hbm.at[gw, sc + 1],
                                      ibuf.at[1 - q], isem[1 - q]).wait()
                pltpu.make_async_copy(ev_hbm.at[gw, sc + 1],
                                      ebuf.at[1 - q], esem[1 - q]).wait()

        if t == 0:
            g_q, g_t = q, 3
        else:
            g_q, g_t = 1 - q, t - 1

        @pl.when(k + 3 < NCHUNK)
        def _():
            pltpu.async_copy(feats_hbm.at[ibuf.at[g_q, 0, g_t]],
                             gbuf.at[(t + 3) % 4], gsem[(t + 3) % 4])

        def scale(g, _):
            ev16 = ebuf[q, t, pl.ds(pl.multiple_of(g * 16, 8), 16)]
            for e in range(16):
                evb = jnp.full((16,), ev16[e], jnp.float32)
                r = g * 16 + e
                for j in range(D // 16):
                    gbuf[t, r, pl.ds(j * 16, 16)] = (
                        gbuf[t, r, pl.ds(j * 16, 16)] * evb)
            return 0
        lax.fori_loop(0, CHUNK // 16, scale, 0)

        pltpu.async_copy(gbuf.at[t], acc.at[ibuf.at[q, 1, t]], ssem[t],
                         add=True)

    def super_pair(scp, _):
        for t in range(G):
            step(2 * scp, 0, t)
        for t in range(G):
            step(2 * scp + 1, 1, t)
        return 0
    lax.fori_loop(0, NSUP // 2, super_pair, 0)

    pltpu.make_async_copy(gbuf.at[3], acc.at[ibuf.at[1, 1, 3]],
                          ssem3).wait()
    plsc.subcore_barrier()

    @pl.when(s < 15)
    def _():
        off = pl.multiple_of(s * ROWS_BASE, 8)
        pltpu.sync_copy(acc.at[pl.ds(off, ROWS_BASE)],
                        out_hbm.at[c, pl.ds(off, ROWS_BASE)])

    @pl.when(s == 15)
    def _():
        off = 15 * ROWS_BASE
        pltpu.sync_copy(acc.at[pl.ds(off, ROWS_LAST)],
                        out_hbm.at[c, pl.ds(off, ROWS_LAST)])


_sc_segment = functools.partial(
    pl.kernel,
    out_type=jax.ShapeDtypeStruct((NC, N, D), jnp.float32),
    mesh=plsc.VectorSubcoreMesh(core_axis_name="c", subcore_axis_name="s"),
    scratch_types=[
        pltpu.VMEM((2, 2, G, CHUNK), jnp.int32),
        pltpu.VMEM((2, G, CHUNK), jnp.float32),
        pltpu.VMEM((4, CHUNK, D), jnp.float32),
        pltpu.VMEM_SHARED((N, D), jnp.float32),
        pltpu.SemaphoreType.DMA,
        pltpu.SemaphoreType.DMA,
        pltpu.SemaphoreType.DMA,
        pltpu.SemaphoreType.DMA,
        pltpu.SemaphoreType.DMA,
        pltpu.SemaphoreType.DMA,
        pltpu.SemaphoreType.DMA,
        pltpu.SemaphoreType.DMA,
        pltpu.SemaphoreType.DMA,
        pltpu.SemaphoreType.DMA,
        pltpu.SemaphoreType.DMA,
        pltpu.SemaphoreType.DMA,
    ],
)(_sc_body)


def _tc_body(lep_ref, f_ref, w1_ref, w2_ref, b1_ref, b2_ref, o_ref):
    le = lep_ref[0] + lep_ref[1]
    f = f_ref[...]
    sf = le + f
    em = le * f
    acc = lax.dot_general(sf, w1_ref[...], (((1,), (1,)), ((), ())),
                          preferred_element_type=jnp.float32)
    acc = acc + lax.dot_general(em, w2_ref[...], (((1,), (1,)), ((), ())),
                                preferred_element_type=jnp.float32)
    o_ref[...] = acc + b1_ref[...] + b2_ref[...]


_BN = 1000


def _tc_dense(lep, feats, W1_w, W1_b, W2_w, W2_b):
    return pl.pallas_call(
        _tc_body,
        grid=(N // _BN,),
        in_specs=[
            pl.BlockSpec((NC, _BN, D), lambda i: (0, i, 0)),
            pl.BlockSpec((_BN, D), lambda i: (i, 0)),
            pl.BlockSpec((D, D), lambda i: (0, 0)),
            pl.BlockSpec((D, D), lambda i: (0, 0)),
            pl.BlockSpec((1, D), lambda i: (0, 0)),
            pl.BlockSpec((1, D), lambda i: (0, 0)),
        ],
        out_specs=pl.BlockSpec((_BN, D), lambda i: (i, 0)),
        out_shape=jax.ShapeDtypeStruct((N, D), jnp.float32),
    )(lep, feats, W1_w, W2_w, W1_b.reshape(1, D), W2_b.reshape(1, D))


def kernel(edge_index, edge_values, feats, W1_w, W1_b, W2_w, W2_b):
    pad = EPAD - E
    spread = (jnp.arange(pad, dtype=jnp.int32) * 8) % N
    src = jnp.concatenate([edge_index[0], spread])
    dst = jnp.concatenate([edge_index[1], spread])
    ev = jnp.concatenate([edge_values, jnp.zeros((pad,), jnp.float32)])
    shp = (NW, NSUP, 1, G, CHUNK)
    combo = jnp.concatenate([src.reshape(shp), dst.reshape(shp)], axis=2)
    evr = ev.reshape(NW, NSUP, G, CHUNK)
    lep = _sc_segment(combo, evr, feats)
    return _tc_dense(lep, feats, W1_w, W1_b, W2_w, W2_b)


# speedup vs baseline: 2.0443x; 1.0149x over previous
import functools

import jax
import jax.numpy as jnp
from jax import lax
from jax.experimental import pallas as pl
from jax.experimental.pallas import tpu as pltpu
from jax.experimental.pallas import tpu_sc as plsc

N = 10000
E = 320000
D = 128

NC = 2
NS = 16
NW = NC * NS
CHUNK = 64
G = 4
EPW = 10240
EPAD = NW * EPW
NCHUNK = EPW // CHUNK
NSUP = NCHUNK // G
ROWS_BASE = 624
ROWS_LAST = N - 15 * ROWS_BASE
NZFULL = N // CHUNK
NZTAIL = N - NZFULL * CHUNK


def _sc_body(src_hbm, dst_hbm, ev_hbm, feats_hbm, out_hbm,
             sbuf_i, dbuf_i, ebuf, gbuf, acc,
             asem0, asem1, bsem0, bsem1, esem0, esem1,
             gsem0, gsem1, gsem2, gsem3, ssem0, ssem1, ssem2, ssem3):
    c = lax.axis_index("c")
    s = lax.axis_index("s")
    gw = c * NS + s
    asem = (asem0, asem1)
    bsem = (bsem0, bsem1)
    esem = (esem0, esem1)
    gsem = (gsem0, gsem1, gsem2, gsem3)
    ssem = (ssem0, ssem1, ssem2, ssem3)

    ld_a = pltpu.async_copy(src_hbm.at[gw, 0], sbuf_i.at[0], asem0)
    pltpu.async_copy(src_hbm.at[gw, 1], sbuf_i.at[1], asem1)
    ld_b = pltpu.async_copy(dst_hbm.at[gw, 0], dbuf_i.at[0], bsem0)
    pltpu.async_copy(dst_hbm.at[gw, 1], dbuf_i.at[1], bsem1)
    ld_e = pltpu.async_copy(ev_hbm.at[gw, 0], ebuf.at[0], esem0)
    pltpu.async_copy(ev_hbm.at[gw, 1], ebuf.at[1], esem1)

    def zrow(i, _):
        for j in range(D // 16):
            gbuf[0, i, pl.ds(j * 16, 16)] = jnp.zeros((16,), jnp.float32)
        return 0
    lax.fori_loop(0, CHUNK, zrow, 0)
    for t in range(10):
        zk = s * 10 + t

        @pl.when(zk < NZFULL)
        def _():
            off = pl.multiple_of(zk * CHUNK, 8)
            pltpu.sync_copy(gbuf.at[0], acc.at[pl.ds(off, CHUNK)])

    @pl.when(s == 15)
    def _():
        pltpu.sync_copy(gbuf.at[0, pl.ds(0, NZTAIL)],
                        acc.at[pl.ds(NZFULL * CHUNK, NZTAIL)])

    ld_a.wait()
    ld_b.wait()
    ld_e.wait()
    pltpu.async_copy(feats_hbm.at[sbuf_i.at[0, 0]], gbuf.at[0], gsem0)
    pltpu.async_copy(feats_hbm.at[sbuf_i.at[0, 1]], gbuf.at[1], gsem1)
    pltpu.async_copy(feats_hbm.at[sbuf_i.at[0, 2]], gbuf.at[2], gsem2)
    plsc.subcore_barrier()

    def step(sc, q, t):
        k = G * sc + t

        pltpu.make_async_copy(feats_hbm.at[sbuf_i.at[q, t]], gbuf.at[t],
                              gsem[t]).wait()

        if t >= 1:
            d_q, d_t = q, t - 1
        else:
            d_q, d_t = 1 - q, 3

        @pl.when(k >= 1)
        def _():
            pltpu.make_async_copy(gbuf.at[(t + 3) % 4],
                                  acc.at[dbuf_i.at[d_q, d_t]],
                                  ssem[(t + 3) % 4]).wait()

        if t == 0:
            @pl.when(jnp.logical_and(sc >= 1, sc + 1 < NSUP))
            def _():
                pltpu.async_copy(src_hbm.at[gw, sc + 1], sbuf_i.at[1 - q],
                                 asem[1 - q])
                pltpu.async_copy(dst_hbm.at[gw, sc + 1], dbuf_i.at[1 - q],
                                 bsem[1 - q])
                pltpu.async_copy(ev_hbm.at[gw, sc + 1], ebuf.at[1 - q],
                                 esem[1 - q])
        if t == 1:
            @pl.when(sc + 1 < NSUP)
            def _():
                pltpu.make_async_copy(src_hbm.at[gw, sc + 1],
                                      sbuf_i.at[1 - q], asem[1 - q]).wait()
                pltpu.make_async_copy(dst_hbm.at[gw, sc + 1],
                                      dbuf_i.at[1 - q], bsem[1 - q]).wait()
                pltpu.make_async_copy(ev_hbm.at[gw, sc + 1],
                                      ebuf.at[1 - q], esem[1 - q]).wait()

        if t == 0:
            g_q, g_t = q, 3
        else:
            g_q, g_t = 1 - q, t - 1

        @pl.when(k + 3 < NCHUNK)
        def _():
            pltpu.async_copy(feats_hbm.at[sbuf_i.at[g_q, g_t]],
                             gbuf.at[(t + 3) % 4], gsem[(t + 3) % 4])

        def scale(g, _):
            ev16 = ebuf[q, t, pl.ds(pl.multiple_of(g * 16, 8), 16)]
            for e in range(16):
                evb = jnp.full((16,), ev16[e], jnp.float32)
                r = g * 16 + e
                for j in range(D // 16):
                    gbuf[t, r, pl.ds(j * 16, 16)] = (
                        gbuf[t, r, pl.ds(j * 16, 16)] * evb)
            return 0
        lax.fori_loop(0, CHUNK // 16, scale, 0)

        pltpu.async_copy(gbuf.at[t], acc.at[dbuf_i.at[q, t]], ssem[t],
                         add=True)

    def super_pair(scp, _):
        for t in range(G):
            step(2 * scp, 0, t)
        for t in range(G):
            step(2 * scp + 1, 1, t)
        return 0
    lax.fori_loop(0, NSUP // 2, super_pair, 0)

    pltpu.make_async_copy(gbuf.at[3], acc.at[dbuf_i.at[1, 3]],
                          ssem3).wait()
    plsc.subcore_barrier()

    @pl.when(s < 15)
    def _():
        off = pl.multiple_of(s * ROWS_BASE, 8)
        pltpu.sync_copy(acc.at[pl.ds(off, ROWS_BASE)],
                        out_hbm.at[c, pl.ds(off, ROWS_BASE)])

    @pl.when(s == 15)
    def _():
        off = 15 * ROWS_BASE
        pltpu.sync_copy(acc.at[pl.ds(off, ROWS_LAST)],
                        out_hbm.at[c, pl.ds(off, ROWS_LAST)])


_sc_segment = functools.partial(
    pl.kernel,
    out_type=jax.ShapeDtypeStruct((NC, N, D), jnp.float32),
    mesh=plsc.VectorSubcoreMesh(core_axis_name="c", subcore_axis_name="s"),
    scratch_types=[
        pltpu.VMEM((2, G, CHUNK), jnp.int32),
        pltpu.VMEM((2, G, CHUNK), jnp.int32),
        pltpu.VMEM((2, G, CHUNK), jnp.float32),
        pltpu.VMEM((4, CHUNK, D), jnp.float32),
        pltpu.VMEM_SHARED((N, D), jnp.float32),
        pltpu.SemaphoreType.DMA,
        pltpu.SemaphoreType.DMA,
        pltpu.SemaphoreType.DMA,
        pltpu.SemaphoreType.DMA,
        pltpu.SemaphoreType.DMA,
        pltpu.SemaphoreType.DMA,
        pltpu.SemaphoreType.DMA,
        pltpu.SemaphoreType.DMA,
        pltpu.SemaphoreType.DMA,
        pltpu.SemaphoreType.DMA,
        pltpu.SemaphoreType.DMA,
        pltpu.SemaphoreType.DMA,
        pltpu.SemaphoreType.DMA,
        pltpu.SemaphoreType.DMA,
    ],
)(_sc_body)


def _tc_body(lep_ref, f_ref, w1_ref, w2_ref, b1_ref, b2_ref, o_ref):
    le = lep_ref[0] + lep_ref[1]
    f = f_ref[...]
    sf = le + f
    em = le * f
    acc = lax.dot_general(sf, w1_ref[...], (((1,), (1,)), ((), ())),
                          preferred_element_type=jnp.float32)
    acc = acc + lax.dot_general(em, w2_ref[...], (((1,), (1,)), ((), ())),
                                preferred_element_type=jnp.float32)
    o_ref[...] = acc + b1_ref[...] + b2_ref[...]


_BN = 1000


def _tc_dense(lep, feats, W1_w, W1_b, W2_w, W2_b):
    return pl.pallas_call(
        _tc_body,
        grid=(N // _BN,),
        in_specs=[
            pl.BlockSpec((NC, _BN, D), lambda i: (0, i, 0)),
            pl.BlockSpec((_BN, D), lambda i: (i, 0)),
            pl.BlockSpec((D, D), lambda i: (0, 0)),
            pl.BlockSpec((D, D), lambda i: (0, 0)),
            pl.BlockSpec((1, D), lambda i: (0, 0)),
            pl.BlockSpec((1, D), lambda i: (0, 0)),
        ],
        out_specs=pl.BlockSpec((_BN, D), lambda i: (i, 0)),
        out_shape=jax.ShapeDtypeStruct((N, D), jnp.float32),
    )(lep, feats, W1_w, W2_w, W1_b.reshape(1, D), W2_b.reshape(1, D))


def kernel(edge_index, edge_values, feats, W1_w, W1_b, W2_w, W2_b):
    pad = EPAD - E
    spread = (jnp.arange(pad, dtype=jnp.int32) * 8) % N
    src = jnp.concatenate([edge_index[0], spread])
    dst = jnp.concatenate([edge_index[1], spread])
    ev = jnp.concatenate([edge_values, jnp.zeros((pad,), jnp.float32)])
    shp = (NW, NSUP, G, CHUNK)
    lep = _sc_segment(src.reshape(shp), dst.reshape(shp), ev.reshape(shp),
                      feats)
    return _tc_dense(lep, feats, W1_w, W1_b, W2_w, W2_b)


# speedup vs baseline: 2.0490x; 1.0023x over previous
import functools

import jax
import jax.numpy as jnp
from jax import lax
from jax.experimental import pallas as pl
from jax.experimental.pallas import tpu as pltpu
from jax.experimental.pallas import tpu_sc as plsc

N = 10000
E = 320000
D = 128

NC = 2
NS = 16
NW = NC * NS
CHUNK = 64
G = 4
EPW = 10240
EPAD = NW * EPW
NCHUNK = EPW // CHUNK
NSUP = NCHUNK // G
ROWS_BASE = 624
ROWS_LAST = N - 15 * ROWS_BASE
NZFULL = N // CHUNK
NZTAIL = N - NZFULL * CHUNK


def _sc_body(src_hbm, dst_hbm, ev_hbm, feats_hbm, out_hbm,
             sbuf_i, dbuf_i, ebuf, gbuf, acc,
             asem0, asem1, bsem0, bsem1, esem0, esem1,
             gsem0, gsem1, gsem2, gsem3, ssem0, ssem1, ssem2, ssem3, zsem):
    c = lax.axis_index("c")
    s = lax.axis_index("s")
    gw = c * NS + s
    asem = (asem0, asem1)
    bsem = (bsem0, bsem1)
    esem = (esem0, esem1)
    gsem = (gsem0, gsem1, gsem2, gsem3)
    ssem = (ssem0, ssem1, ssem2, ssem3)

    ld_a = pltpu.async_copy(src_hbm.at[gw, 0], sbuf_i.at[0], asem0)
    pltpu.async_copy(src_hbm.at[gw, 1], sbuf_i.at[1], asem1)
    ld_b = pltpu.async_copy(dst_hbm.at[gw, 0], dbuf_i.at[0], bsem0)
    pltpu.async_copy(dst_hbm.at[gw, 1], dbuf_i.at[1], bsem1)
    ld_e = pltpu.async_copy(ev_hbm.at[gw, 0], ebuf.at[0], esem0)
    pltpu.async_copy(ev_hbm.at[gw, 1], ebuf.at[1], esem1)

    def zrow(i, _):
        for j in range(D // 16):
            gbuf[0, i, pl.ds(j * 16, 16)] = jnp.zeros((16,), jnp.float32)
        return 0
    lax.fori_loop(0, CHUNK, zrow, 0)
    for t in range(10):
        zk = s * 10 + t

        @pl.when(zk < NZFULL)
        def _():
            off = pl.multiple_of(zk * CHUNK, 8)
            pltpu.async_copy(gbuf.at[0], acc.at[pl.ds(off, CHUNK)], zsem)

    @pl.when(s == 15)
    def _():
        pltpu.async_copy(gbuf.at[0, pl.ds(0, NZTAIL)],
                        acc.at[pl.ds(NZFULL * CHUNK, NZTAIL)], zsem)

    for t in range(10):
        zk = s * 10 + t

        @pl.when(zk < NZFULL)
        def _():
            off = pl.multiple_of(zk * CHUNK, 8)
            pltpu.make_async_copy(gbuf.at[0], acc.at[pl.ds(off, CHUNK)],
                                  zsem).wait()

    @pl.when(s == 15)
    def _():
        pltpu.make_async_copy(gbuf.at[0, pl.ds(0, NZTAIL)],
                              acc.at[pl.ds(NZFULL * CHUNK, NZTAIL)],
                              zsem).wait()

    ld_a.wait()
    ld_b.wait()
    ld_e.wait()
    pltpu.async_copy(feats_hbm.at[sbuf_i.at[0, 0]], gbuf.at[0], gsem0)
    pltpu.async_copy(feats_hbm.at[sbuf_i.at[0, 1]], gbuf.at[1], gsem1)
    pltpu.async_copy(feats_hbm.at[sbuf_i.at[0, 2]], gbuf.at[2], gsem2)
    plsc.subcore_barrier()

    def step(sc, q, t):
        k = G * sc + t

        pltpu.make_async_copy(feats_hbm.at[sbuf_i.at[q, t]], gbuf.at[t],
                              gsem[t]).wait()

        if t >= 1:
            d_q, d_t = q, t - 1
        else:
            d_q, d_t = 1 - q, 3

        @pl.when(k >= 1)
        def _():
            pltpu.make_async_copy(gbuf.at[(t + 3) % 4],
                                  acc.at[dbuf_i.at[d_q, d_t]],
                                  ssem[(t + 3) % 4]).wait()

        if t == 0:
            @pl.when(jnp.logical_and(sc >= 1, sc + 1 < NSUP))
            def _():
                pltpu.async_copy(src_hbm.at[gw, sc + 1], sbuf_i.at[1 - q],
                                 asem[1 - q])
                pltpu.async_copy(dst_hbm.at[gw, sc + 1], dbuf_i.at[1 - q],
                                 bsem[1 - q])
                pltpu.async_copy(ev_hbm.at[gw, sc + 1], ebuf.at[1 - q],
                                 esem[1 - q])
        if t == 1:
            @pl.when(sc + 1 < NSUP)
            def _():
                pltpu.make_async_copy(src_hbm.at[gw, sc + 1],
                                      sbuf_i.at[1 - q], asem[1 - q]).wait()
                pltpu.make_async_copy(dst_hbm.at[gw, sc + 1],
                                      dbuf_i.at[1 - q], bsem[1 - q]).wait()
                pltpu.make_async_copy(ev_hbm.at[gw, sc + 1],
                                      ebuf.at[1 - q], esem[1 - q]).wait()

        if t == 0:
            g_q, g_t = q, 3
        else:
            g_q, g_t = 1 - q, t - 1

        @pl.when(k + 3 < NCHUNK)
        def _():
            pltpu.async_copy(feats_hbm.at[sbuf_i.at[g_q, g_t]],
                             gbuf.at[(t + 3) % 4], gsem[(t + 3) % 4])

        def scale(g, _):
            ev16 = ebuf[q, t, pl.ds(pl.multiple_of(g * 16, 8), 16)]
            for e in range(16):
                evb = jnp.full((16,), ev16[e], jnp.float32)
                r = g * 16 + e
                for j in range(D // 16):
                    gbuf[t, r, pl.ds(j * 16, 16)] = (
                        gbuf[t, r, pl.ds(j * 16, 16)] * evb)
            return 0
        lax.fori_loop(0, CHUNK // 16, scale, 0)

        pltpu.async_copy(gbuf.at[t], acc.at[dbuf_i.at[q, t]], ssem[t],
                         add=True)

    def super_pair(scp, _):
        for t in range(G):
            step(2 * scp, 0, t)
        for t in range(G):
            step(2 * scp + 1, 1, t)
        return 0
    lax.fori_loop(0, NSUP // 2, super_pair, 0)

    pltpu.make_async_copy(gbuf.at[3], acc.at[dbuf_i.at[1, 3]],
                          ssem3).wait()
    plsc.subcore_barrier()

    @pl.when(s < 15)
    def _():
        off = pl.multiple_of(s * ROWS_BASE, 8)
        pltpu.sync_copy(acc.at[pl.ds(off, ROWS_BASE)],
                        out_hbm.at[c, pl.ds(off, ROWS_BASE)])

    @pl.when(s == 15)
    def _():
        off = 15 * ROWS_BASE
        pltpu.sync_copy(acc.at[pl.ds(off, ROWS_LAST)],
                        out_hbm.at[c, pl.ds(off, ROWS_LAST)])


_sc_segment = functools.partial(
    pl.kernel,
    out_type=jax.ShapeDtypeStruct((NC, N, D), jnp.float32),
    mesh=plsc.VectorSubcoreMesh(core_axis_name="c", subcore_axis_name="s"),
    scratch_types=[
        pltpu.VMEM((2, G, CHUNK), jnp.int32),
        pltpu.VMEM((2, G, CHUNK), jnp.int32),
        pltpu.VMEM((2, G, CHUNK), jnp.float32),
        pltpu.VMEM((4, CHUNK, D), jnp.float32),
        pltpu.VMEM_SHARED((N, D), jnp.float32),
        pltpu.SemaphoreType.DMA,
        pltpu.SemaphoreType.DMA,
        pltpu.SemaphoreType.DMA,
        pltpu.SemaphoreType.DMA,
        pltpu.SemaphoreType.DMA,
        pltpu.SemaphoreType.DMA,
        pltpu.SemaphoreType.DMA,
        pltpu.SemaphoreType.DMA,
        pltpu.SemaphoreType.DMA,
        pltpu.SemaphoreType.DMA,
        pltpu.SemaphoreType.DMA,
        pltpu.SemaphoreType.DMA,
        pltpu.SemaphoreType.DMA,
        pltpu.SemaphoreType.DMA,
        pltpu.SemaphoreType.DMA,
    ],
)(_sc_body)


def _tc_body(lep_ref, f_ref, w1_ref, w2_ref, b1_ref, b2_ref, o_ref):
    le = lep_ref[0] + lep_ref[1]
    f = f_ref[...]
    sf = le + f
    em = le * f
    acc = lax.dot_general(sf, w1_ref[...], (((1,), (1,)), ((), ())),
                          preferred_element_type=jnp.float32)
    acc = acc + lax.dot_general(em, w2_ref[...], (((1,), (1,)), ((), ())),
                                preferred_element_type=jnp.float32)
    o_ref[...] = acc + b1_ref[...] + b2_ref[...]


_BN = 1000


def _tc_dense(lep, feats, W1_w, W1_b, W2_w, W2_b):
    return pl.pallas_call(
        _tc_body,
        grid=(N // _BN,),
        in_specs=[
            pl.BlockSpec((NC, _BN, D), lambda i: (0, i, 0)),
            pl.BlockSpec((_BN, D), lambda i: (i, 0)),
            pl.BlockSpec((D, D), lambda i: (0, 0)),
            pl.BlockSpec((D, D), lambda i: (0, 0)),
            pl.BlockSpec((1, D), lambda i: (0, 0)),
            pl.BlockSpec((1, D), lambda i: (0, 0)),
        ],
        out_specs=pl.BlockSpec((_BN, D), lambda i: (i, 0)),
        out_shape=jax.ShapeDtypeStruct((N, D), jnp.float32),
    )(lep, feats, W1_w, W2_w, W1_b.reshape(1, D), W2_b.reshape(1, D))


def kernel(edge_index, edge_values, feats, W1_w, W1_b, W2_w, W2_b):
    pad = EPAD - E
    spread = (jnp.arange(pad, dtype=jnp.int32) * 8) % N
    src = jnp.concatenate([edge_index[0], spread])
    dst = jnp.concatenate([edge_index[1], spread])
    ev = jnp.concatenate([edge_values, jnp.zeros((pad,), jnp.float32)])
    shp = (NW, NSUP, G, CHUNK)
    lep = _sc_segment(src.reshape(shp), dst.reshape(shp), ev.reshape(shp),
                      feats)
    return _tc_dense(lep, feats, W1_w, W1_b, W2_w, W2_b)


# speedup vs baseline: 2.0881x; 1.0191x over previous
import functools

import jax
import jax.numpy as jnp
from jax import lax
from jax.experimental import pallas as pl
from jax.experimental.pallas import tpu as pltpu
from jax.experimental.pallas import tpu_sc as plsc

N = 10000
E = 320000
D = 128

NC = 2
NS = 16
NW = NC * NS
CHUNK = 64
G = 4
EPW = 10240
EPAD = NW * EPW
NCHUNK = EPW // CHUNK
NSUP = NCHUNK // G
ROWS_BASE = 624
ROWS_LAST = N - 15 * ROWS_BASE
NZFULL = N // CHUNK
NZTAIL = N - NZFULL * CHUNK


def _sc_body(src_hbm, dst_hbm, ev_hbm, feats_hbm, out_hbm,
             sbuf_i, dbuf_i, ebuf, gbuf, acc,
             asem0, asem1, bsem0, bsem1, esem0, esem1,
             gsem0, gsem1, gsem2, gsem3, ssem0, ssem1, ssem2, ssem3, zsem):
    c = lax.axis_index("c")
    s = lax.axis_index("s")
    gw = c * NS + s
    asem = (asem0, asem1)
    bsem = (bsem0, bsem1)
    esem = (esem0, esem1)
    gsem = (gsem0, gsem1, gsem2, gsem3)
    ssem = (ssem0, ssem1, ssem2, ssem3)

    ld_a = pltpu.async_copy(src_hbm.at[gw, 0], sbuf_i.at[0], asem0)
    pltpu.async_copy(src_hbm.at[gw, 1], sbuf_i.at[1], asem1)
    ld_b = pltpu.async_copy(dst_hbm.at[gw, 0], dbuf_i.at[0], bsem0)
    pltpu.async_copy(dst_hbm.at[gw, 1], dbuf_i.at[1], bsem1)
    ld_e = pltpu.async_copy(ev_hbm.at[gw, 0], ebuf.at[0], esem0)
    pltpu.async_copy(ev_hbm.at[gw, 1], ebuf.at[1], esem1)

    def zrow(i, _):
        for j in range(D // 16):
            gbuf[0, i, pl.ds(j * 16, 16)] = jnp.zeros((16,), jnp.float32)
        return 0
    lax.fori_loop(0, CHUNK, zrow, 0)
    for t in range(10):
        zk = s * 10 + t

        @pl.when(zk < NZFULL)
        def _():
            off = pl.multiple_of(zk * CHUNK, 8)
            pltpu.async_copy(gbuf.at[0], acc.at[pl.ds(off, CHUNK)], zsem)

    @pl.when(s == 15)
    def _():
        pltpu.async_copy(gbuf.at[0, pl.ds(0, NZTAIL)],
                        acc.at[pl.ds(NZFULL * CHUNK, NZTAIL)], zsem)

    for t in range(10):
        zk = s * 10 + t

        @pl.when(zk < NZFULL)
        def _():
            off = pl.multiple_of(zk * CHUNK, 8)
            pltpu.make_async_copy(gbuf.at[0], acc.at[pl.ds(off, CHUNK)],
                                  zsem).wait()

    @pl.when(s == 15)
    def _():
        pltpu.make_async_copy(gbuf.at[0, pl.ds(0, NZTAIL)],
                              acc.at[pl.ds(NZFULL * CHUNK, NZTAIL)],
                              zsem).wait()

    ld_a.wait()
    ld_b.wait()
    ld_e.wait()
    pltpu.async_copy(feats_hbm.at[sbuf_i.at[0, 0]], gbuf.at[0], gsem0)
    pltpu.async_copy(feats_hbm.at[sbuf_i.at[0, 1]], gbuf.at[1], gsem1)
    pltpu.async_copy(feats_hbm.at[sbuf_i.at[0, 2]], gbuf.at[2], gsem2)
    plsc.subcore_barrier()

    def step(sc, q, t):
        k = G * sc + t

        pltpu.make_async_copy(feats_hbm.at[sbuf_i.at[q, t]], gbuf.at[t],
                              gsem[t]).wait()

        if t >= 1:
            d_q, d_t = q, t - 1
        else:
            d_q, d_t = 1 - q, 3

        @pl.when(k >= 1)
        def _():
            pltpu.make_async_copy(gbuf.at[(t + 3) % 4],
                                  acc.at[dbuf_i.at[d_q, d_t]],
                                  ssem[(t + 3) % 4]).wait()

        if t == 0:
            @pl.when(jnp.logical_and(sc >= 1, sc + 1 < NSUP))
            def _():
                pltpu.async_copy(src_hbm.at[gw, sc + 1], sbuf_i.at[1 - q],
                                 asem[1 - q])
                pltpu.async_copy(dst_hbm.at[gw, sc + 1], dbuf_i.at[1 - q],
                                 bsem[1 - q])
                pltpu.async_copy(ev_hbm.at[gw, sc + 1], ebuf.at[1 - q],
                                 esem[1 - q])
        if t == 1:
            @pl.when(sc + 1 < NSUP)
            def _():
                pltpu.make_async_copy(src_hbm.at[gw, sc + 1],
                                      sbuf_i.at[1 - q], asem[1 - q]).wait()
                pltpu.make_async_copy(dst_hbm.at[gw, sc + 1],
                                      dbuf_i.at[1 - q], bsem[1 - q]).wait()
                pltpu.make_async_copy(ev_hbm.at[gw, sc + 1],
                                      ebuf.at[1 - q], esem[1 - q]).wait()

        if t == 0:
            g_q, g_t = q, 3
        else:
            g_q, g_t = 1 - q, t - 1

        @pl.when(k + 3 < NCHUNK)
        def _():
            pltpu.async_copy(feats_hbm.at[sbuf_i.at[g_q, g_t]],
                             gbuf.at[(t + 3) % 4], gsem[(t + 3) % 4])

        def scale(g, _):
            ev16 = ebuf[q, t, pl.ds(pl.multiple_of(g * 16, 8), 16)]
            for e in range(16):
                evb = jnp.full((16,), ev16[e], jnp.float32)
                r = g * 16 + e
                for j in range(D // 16):
                    gbuf[t, r, pl.ds(j * 16, 16)] = (
                        gbuf[t, r, pl.ds(j * 16, 16)] * evb)
            return 0
        lax.fori_loop(0, CHUNK // 16, scale, 0)

        pltpu.async_copy(gbuf.at[t], acc.at[dbuf_i.at[q, t]], ssem[t],
                         add=True)

    def super_pair(scp, _):
        for t in range(G):
            step(2 * scp, 0, t)
        for t in range(G):
            step(2 * scp + 1, 1, t)
        return 0
    lax.fori_loop(0, NSUP // 2, super_pair, 0)

    pltpu.make_async_copy(gbuf.at[3], acc.at[dbuf_i.at[1, 3]],
                          ssem3).wait()
    plsc.subcore_barrier()

    @pl.when(s < 15)
    def _():
        off = pl.multiple_of(s * ROWS_BASE, 8)
        pltpu.sync_copy(acc.at[pl.ds(off, ROWS_BASE)],
                        out_hbm.at[c, pl.ds(off, ROWS_BASE)])

    @pl.when(s == 15)
    def _():
        off = 15 * ROWS_BASE
        pltpu.sync_copy(acc.at[pl.ds(off, ROWS_LAST)],
                        out_hbm.at[c, pl.ds(off, ROWS_LAST)])


_sc_segment = functools.partial(
    pl.kernel,
    out_type=jax.ShapeDtypeStruct((NC, N, D), jnp.float32),
    mesh=plsc.VectorSubcoreMesh(core_axis_name="c", subcore_axis_name="s"),
    scratch_types=[
        pltpu.VMEM((2, G, CHUNK), jnp.int32),
        pltpu.VMEM((2, G, CHUNK), jnp.int32),
        pltpu.VMEM((2, G, CHUNK), jnp.float32),
        pltpu.VMEM((4, CHUNK, D), jnp.float32),
        pltpu.VMEM_SHARED((N, D), jnp.float32),
        pltpu.SemaphoreType.DMA,
        pltpu.SemaphoreType.DMA,
        pltpu.SemaphoreType.DMA,
        pltpu.SemaphoreType.DMA,
        pltpu.SemaphoreType.DMA,
        pltpu.SemaphoreType.DMA,
        pltpu.SemaphoreType.DMA,
        pltpu.SemaphoreType.DMA,
        pltpu.SemaphoreType.DMA,
        pltpu.SemaphoreType.DMA,
        pltpu.SemaphoreType.DMA,
        pltpu.SemaphoreType.DMA,
        pltpu.SemaphoreType.DMA,
        pltpu.SemaphoreType.DMA,
        pltpu.SemaphoreType.DMA,
    ],
)(_sc_body)


def _tc_body(lep_ref, f_ref, w1_ref, w2_ref, b1_ref, b2_ref, o_ref):
    le = lep_ref[0] + lep_ref[1]
    f = f_ref[...]
    sf = le + f
    em = le * f
    acc = lax.dot_general(sf, w1_ref[...], (((1,), (1,)), ((), ())),
                          preferred_element_type=jnp.float32)
    acc = acc + lax.dot_general(em, w2_ref[...], (((1,), (1,)), ((), ())),
                                preferred_element_type=jnp.float32)
    o_ref[...] = acc + b1_ref[...] + b2_ref[...]


_BN = 2000


def _tc_dense(lep, feats, W1_w, W1_b, W2_w, W2_b):
    return pl.pallas_call(
        _tc_body,
        grid=(N // _BN,),
        in_specs=[
            pl.BlockSpec((NC, _BN, D), lambda i: (0, i, 0)),
            pl.BlockSpec((_BN, D), lambda i: (i, 0)),
            pl.BlockSpec((D, D), lambda i: (0, 0)),
            pl.BlockSpec((D, D), lambda i: (0, 0)),
            pl.BlockSpec((1, D), lambda i: (0, 0)),
            pl.BlockSpec((1, D), lambda i: (0, 0)),
        ],
        out_specs=pl.BlockSpec((_BN, D), lambda i: (i, 0)),
        out_shape=jax.ShapeDtypeStruct((N, D), jnp.float32),
    )(lep, feats, W1_w, W2_w, W1_b.reshape(1, D), W2_b.reshape(1, D))


def kernel(edge_index, edge_values, feats, W1_w, W1_b, W2_w, W2_b):
    pad = EPAD - E
    spread = (jnp.arange(pad, dtype=jnp.int32) * 8) % N
    src = jnp.concatenate([edge_index[0], spread])
    dst = jnp.concatenate([edge_index[1], spread])
    ev = jnp.concatenate([edge_values, jnp.zeros((pad,), jnp.float32)])
    shp = (NW, NSUP, G, CHUNK)
    lep = _sc_segment(src.reshape(shp), dst.reshape(shp), ev.reshape(shp),
                      feats)
    return _tc_dense(lep, feats, W1_w, W1_b, W2_w, W2_b)
